# baseline (speedup 1.0000x reference)
import functools

import jax
import jax.numpy as jnp
from jax import lax
from jax.experimental import pallas as pl
from jax.experimental.pallas import tpu as pltpu
from jax.experimental.pallas import tpu_sc as plsc

_NC = 2
_NS = 16
_HEAD = 256


def _sc_gather_rows(idx, table, off, num, ch):
    E = table.shape[1]
    NW = _NC * _NS
    b_per_w = num // NW
    n_ch = b_per_w // ch
    mesh = plsc.VectorSubcoreMesh(core_axis_name="c", subcore_axis_name="s")

    @functools.partial(
        pl.kernel,
        out_type=jax.ShapeDtypeStruct((num, E), jnp.float32),
        mesh=mesh,
        scratch_types=[
            pltpu.VMEM((b_per_w,), jnp.int32),
            pltpu.VMEM((ch, E), jnp.float32),
            pltpu.VMEM((ch, E), jnp.float32),
            pltpu.SemaphoreType.DMA,
            pltpu.SemaphoreType.DMA,
            pltpu.SemaphoreType.DMA,
            pltpu.SemaphoreType.DMA,
        ],
    )
    def gather_kernel(idx_hbm, tab_hbm, out_hbm, idx_v, buf0, buf1,
                      g0, g1, s0, s1):
        wid = lax.axis_index("s") * _NC + lax.axis_index("c")
        base = wid * b_per_w
        pltpu.sync_copy(idx_hbm.at[pl.ds(off + base, b_per_w)], idx_v)

        bufs = (buf0, buf1)
        gsems = (g0, g1)
        ssems = (s0, s1)

        def start_gather(c):
            b = c & 1
            return pltpu.async_copy(
                tab_hbm.at[idx_v.at[pl.ds(c * ch, ch)]], bufs[b], gsems[b])

        gat = [None, None]
        scat = [None, None]
        gat[0] = start_gather(0)
        for c in range(n_ch):
            b = c & 1
            nb = (c + 1) & 1
            gat[b].wait()
            if c + 1 < n_ch:
                if scat[nb] is not None:
                    scat[nb].wait()
                gat[nb] = start_gather(c + 1)
            scat[b] = pltpu.make_async_copy(
                bufs[b], out_hbm.at[pl.ds(base + c * ch, ch)], ssems[b])
            scat[b].start()
        scat[0].wait()
        if n_ch > 1:
            scat[1].wait()

    return gather_kernel(idx, table)


def _tc_matmul_pad(A, emd, dep, bb):
    N, K = A.shape
    D = emd.shape[1]

    def body(a_ref, e_ref, d_ref, o_ref):
        del d_ref
        o_ref[:, :D] = jnp.dot(a_ref[...], e_ref[...],
                               preferred_element_type=jnp.float32)
        o_ref[:, D:] = jnp.zeros((a_ref.shape[0], 128 - D), jnp.float32)

    return pl.pallas_call(
        body,
        grid=(N // bb,),
        in_specs=[
            pl.BlockSpec((bb, K), lambda i: (i, 0)),
            pl.BlockSpec((K, D), lambda i: (0, 0)),
            pl.BlockSpec((8, K), lambda i: (0, 0)),
        ],
        out_specs=pl.BlockSpec((bb, 128), lambda i: (i, 0)),
        out_shape=jax.ShapeDtypeStruct((N, 128), jnp.float32),
    )(A, emd, dep)


def _tc_score(Ug, emd, iemb, bb, iemb_off):
    B, E = Ug.shape
    D = emd.shape[1]

    def body(ug_ref, e_ref, ie_ref, o_ref):
        pu = jnp.dot(ug_ref[...], e_ref[...],
                     preferred_element_type=jnp.float32)
        s = jnp.sum(pu * ie_ref[:, :D], axis=1)
        o_ref[...] = jax.nn.sigmoid(s)

    return pl.pallas_call(
        body,
        grid=(B // bb,),
        in_specs=[
            pl.BlockSpec((bb, E), lambda i: (i, 0)),
            pl.BlockSpec((E, D), lambda i: (0, 0)),
            pl.BlockSpec((bb, 128), lambda i: (i + iemb_off, 0)),
        ],
        out_specs=pl.BlockSpec((bb,), lambda i: (i,)),
        out_shape=jax.ShapeDtypeStruct((B,), jnp.float32),
    )(Ug, emd, iemb)


def kernel(users, items, enti_emd, UEnet, IEnet):
    B = users.shape[0]
    Ug_head = _sc_gather_rows(users, UEnet, 0, _HEAD, ch=8)
    Ug_tail = _sc_gather_rows(users, UEnet, _HEAD, B - _HEAD, ch=8)
    item_emd = _tc_matmul_pad(IEnet, enti_emd, Ug_head, bb=512)
    items_emb = _sc_gather_rows(items, item_emd, 0, B, ch=128)
    s_head = _tc_score(Ug_head, enti_emd, items_emb, bb=_HEAD, iemb_off=0)
    s_tail = _tc_score(Ug_tail, enti_emd, items_emb, bb=256, iemb_off=1)
    return jnp.concatenate([s_head, s_tail])


# speedup vs baseline: 1.2059x; 1.2059x over previous
import functools

import jax
import jax.numpy as jnp
from jax import lax
from jax.experimental import pallas as pl
from jax.experimental.pallas import tpu as pltpu
from jax.experimental.pallas import tpu_sc as plsc

_NC = 2
_NS = 16
_CH = 8
_NBUF = 3


def _sc_gather_pair(users, items, UEnet, IEnet):
    B = users.shape[0]
    E = UEnet.shape[1]
    NW = _NC * _NS
    b_per_w = B // NW
    n_ch = b_per_w // _CH
    mesh = plsc.VectorSubcoreMesh(core_axis_name="c", subcore_axis_name="s")

    @functools.partial(
        pl.kernel,
        out_type=(
            jax.ShapeDtypeStruct((B, E), jnp.float32),
            jax.ShapeDtypeStruct((B, E), jnp.float32),
        ),
        mesh=mesh,
        scratch_types=(
            [pltpu.VMEM((b_per_w,), jnp.int32)] * 2
            + [pltpu.VMEM((_CH, E), jnp.float32)] * _NBUF
            + [pltpu.SemaphoreType.DMA] * (2 * _NBUF)
        ),
    )
    def gather_kernel(users_hbm, items_hbm, ue_hbm, ie_hbm, ug_out, ig_out,
                      uidx_v, iidx_v, *bufs_sems):
        bufs = bufs_sems[:_NBUF]
        gsems = bufs_sems[_NBUF:2 * _NBUF]
        ssems = bufs_sems[2 * _NBUF:]
        wid = lax.axis_index("s") * _NC + lax.axis_index("c")
        base = wid * b_per_w
        pltpu.sync_copy(users_hbm.at[pl.ds(base, b_per_w)], uidx_v)
        pltpu.sync_copy(items_hbm.at[pl.ds(base, b_per_w)], iidx_v)

        chunks = []
        for c in range(n_ch):
            chunks.append((ue_hbm, uidx_v, ug_out, c * _CH))
            chunks.append((ie_hbm, iidx_v, ig_out, c * _CH))
        n = len(chunks)

        def start_gather(c):
            tab, idxr, _, off = chunks[c]
            b = c % _NBUF
            return pltpu.async_copy(tab.at[idxr.at[pl.ds(off, _CH)]],
                                    bufs[b], gsems[b])

        gat = [None] * _NBUF
        scat = [None] * _NBUF
        for c in range(min(_NBUF, n)):
            gat[c % _NBUF] = start_gather(c)
        for c in range(n):
            b = c % _NBUF
            gat[b].wait()
            _, _, outr, off = chunks[c]
            scat[b] = pltpu.make_async_copy(
                bufs[b], outr.at[pl.ds(base + off, _CH)], ssems[b])
            scat[b].start()
            if c + _NBUF < n:
                scat[b].wait()
                gat[b] = start_gather(c + _NBUF)
        for b in range(min(_NBUF, n)):
            if scat[b] is not None:
                scat[b].wait()

    return gather_kernel(users, items, UEnet, IEnet)


def _tc_score(Ug, Ig, emd, bb):
    B, E = Ug.shape
    D = emd.shape[1]

    def body(ug_ref, ig_ref, e_ref, o_ref):
        pu = jnp.dot(ug_ref[...], e_ref[...],
                     preferred_element_type=jnp.float32)
        pi = jnp.dot(ig_ref[...], e_ref[...],
                     preferred_element_type=jnp.float32)
        s = jnp.sum(pu * pi, axis=1)
        o_ref[...] = jax.nn.sigmoid(s)

    return pl.pallas_call(
        body,
        grid=(B // bb,),
        in_specs=[
            pl.BlockSpec((bb, E), lambda i: (i, 0)),
            pl.BlockSpec((bb, E), lambda i: (i, 0)),
            pl.BlockSpec((E, D), lambda i: (0, 0)),
        ],
        out_specs=pl.BlockSpec((bb,), lambda i: (i,)),
        out_shape=jax.ShapeDtypeStruct((B,), jnp.float32),
    )(Ug, Ig, emd)


def kernel(users, items, enti_emd, UEnet, IEnet):
    B = users.shape[0]
    h = B // 2
    outs = []
    pairs = []
    for k in range(2):
        u = lax.slice_in_dim(users, k * h, (k + 1) * h)
        it = lax.slice_in_dim(items, k * h, (k + 1) * h)
        pairs.append(_sc_gather_pair(u, it, UEnet, IEnet))
    for Ug, Ig in pairs:
        outs.append(_tc_score(Ug, Ig, enti_emd, bb=512))
    return jnp.concatenate(outs)


# speedup vs baseline: 1.2173x; 1.0094x over previous
import functools

import jax
import jax.numpy as jnp
from jax import lax
from jax.experimental import pallas as pl
from jax.experimental.pallas import tpu as pltpu
from jax.experimental.pallas import tpu_sc as plsc

_NC = 2
_NS = 16
_CH = 8
_NBUF = 3


def _sc_gather_pair(users, items, UEnet, IEnet):
    B = users.shape[0]
    E = UEnet.shape[1]
    NW = _NC * _NS
    b_per_w = B // NW
    n_ch = b_per_w // _CH
    mesh = plsc.VectorSubcoreMesh(core_axis_name="c", subcore_axis_name="s")

    @functools.partial(
        pl.kernel,
        out_type=(
            jax.ShapeDtypeStruct((B, E), jnp.float32),
            jax.ShapeDtypeStruct((B, E), jnp.float32),
        ),
        mesh=mesh,
        scratch_types=(
            [pltpu.VMEM((b_per_w,), jnp.int32)] * 2
            + [pltpu.VMEM((_CH, E), jnp.float32)] * _NBUF
            + [pltpu.SemaphoreType.DMA] * (2 * _NBUF)
        ),
    )
    def gather_kernel(users_hbm, items_hbm, ue_hbm, ie_hbm, ug_out, ig_out,
                      uidx_v, iidx_v, *bufs_sems):
        bufs = bufs_sems[:_NBUF]
        gsems = bufs_sems[_NBUF:2 * _NBUF]
        ssems = bufs_sems[2 * _NBUF:]
        wid = lax.axis_index("s") * _NC + lax.axis_index("c")
        base = wid * b_per_w
        pltpu.sync_copy(users_hbm.at[pl.ds(base, b_per_w)], uidx_v)
        pltpu.sync_copy(items_hbm.at[pl.ds(base, b_per_w)], iidx_v)

        chunks = []
        for c in range(n_ch):
            chunks.append((ue_hbm, uidx_v, ug_out, c * _CH))
            chunks.append((ie_hbm, iidx_v, ig_out, c * _CH))
        n = len(chunks)

        def start_gather(c):
            tab, idxr, _, off = chunks[c]
            b = c % _NBUF
            return pltpu.async_copy(tab.at[idxr.at[pl.ds(off, _CH)]],
                                    bufs[b], gsems[b])

        gat = [None] * _NBUF
        scat = [None] * _NBUF
        for c in range(min(_NBUF, n)):
            gat[c % _NBUF] = start_gather(c)
        for c in range(n):
            b = c % _NBUF
            gat[b].wait()
            _, _, outr, off = chunks[c]
            scat[b] = pltpu.make_async_copy(
                bufs[b], outr.at[pl.ds(base + off, _CH)], ssems[b])
            scat[b].start()
            if c + _NBUF < n:
                scat[b].wait()
                gat[b] = start_gather(c + _NBUF)
        for b in range(min(_NBUF, n)):
            if scat[b] is not None:
                scat[b].wait()

    return gather_kernel(users, items, UEnet, IEnet)


def _tc_score(Ug, Ig, emd, bb):
    B, E = Ug.shape
    D = emd.shape[1]

    def body(ug_ref, ig_ref, e_ref, o_ref):
        pu = jnp.dot(ug_ref[...], e_ref[...],
                     preferred_element_type=jnp.float32)
        pi = jnp.dot(ig_ref[...], e_ref[...],
                     preferred_element_type=jnp.float32)
        s = jnp.sum(pu * pi, axis=1)
        o_ref[...] = jax.nn.sigmoid(s)

    return pl.pallas_call(
        body,
        grid=(B // bb,),
        in_specs=[
            pl.BlockSpec((bb, E), lambda i: (i, 0)),
            pl.BlockSpec((bb, E), lambda i: (i, 0)),
            pl.BlockSpec((E, D), lambda i: (0, 0)),
        ],
        out_specs=pl.BlockSpec((bb,), lambda i: (i,)),
        out_shape=jax.ShapeDtypeStruct((B,), jnp.float32),
    )(Ug, Ig, emd)


def kernel(users, items, enti_emd, UEnet, IEnet):
    Ug, Ig = _sc_gather_pair(users, items, UEnet, IEnet)
    return _tc_score(Ug, Ig, enti_emd, bb=512)
